# initial kernel scaffold (unmeasured)
import functools

import jax
import jax.numpy as jnp
import numpy as np
from jax import lax
from jax.experimental import pallas as pl
from jax.experimental.pallas import tpu as pltpu

N_DEV = 4
B_LOC = 2
SQ = 512
D = 1024
H_LOC = 8
DH = 128
SCALE = 0.08838834764831843


def _rope_consts():
    inv = 1.0 / (10000.0 ** (np.arange(0, DH, 2) / DH))
    pos = np.arange(SQ)[:, None] * inv[None, :]
    cos = np.repeat(np.cos(pos), 2, axis=-1).astype(np.float32)
    sin = np.repeat(np.sin(pos), 2, axis=-1).astype(np.float32)
    R = np.zeros((DH, DH), np.float32)
    for k in range(DH // 2):
        R[2 * k + 1, 2 * k] = -1.0
        R[2 * k, 2 * k + 1] = 1.0
    return jnp.asarray(cos), jnp.asarray(sin), jnp.asarray(R)


def kernel(x, Wq, Wk, Wv, Wo):
    cos, sin, R = _rope_consts()

    def body(x_ref, wq_ref, wk_ref, wv_ref, wo_ref, cos_ref, sin_ref, r_ref,
             out_ref, xbuf, rs_recv, rs_send,
             ag_send_sems, ag_recv_sems, rs_send_sems, rs_recv_sems):
        my = lax.axis_index("i")
        left = (my + N_DEV - 1) % N_DEV
        right = (my + 1) % N_DEV

        barrier_sem = pltpu.get_barrier_semaphore()
        for nbr in (left, right):
            pl.semaphore_signal(
                barrier_sem, inc=1,
                device_id=(nbr,), device_id_type=pl.DeviceIdType.MESH,
            )
        pl.semaphore_wait(barrier_sem, 2)

        cosv = cos_ref[...]
        sinv = sin_ref[...]
        rmat = r_ref[...]
        wq = wq_ref[...]
        wk = wk_ref[...]
        wv = wv_ref[...]
        wo = wo_ref[...]

        def rope(t):
            return t * cosv + jnp.dot(t, rmat,
                                      preferred_element_type=jnp.float32) * sinv

        def compute_partial(xc):
            outs = []
            for b in range(B_LOC):
                xb = xc[b]
                q = jnp.dot(xb, wq, preferred_element_type=jnp.float32)
                k = jnp.dot(xb, wk, preferred_element_type=jnp.float32)
                v = jnp.dot(xb, wv, preferred_element_type=jnp.float32)
                ctx_heads = []
                for h in range(H_LOC):
                    sl = slice(h * DH, (h + 1) * DH)
                    qh = rope(q[:, sl])
                    kh = rope(k[:, sl])
                    vh = v[:, sl]
                    s = lax.dot_general(
                        qh, kh, (((1,), (1,)), ((), ())),
                        preferred_element_type=jnp.float32,
                    ) * SCALE
                    s = s - jnp.max(s, axis=-1, keepdims=True)
                    w = jnp.exp(s)
                    w = w / jnp.sum(w, axis=-1, keepdims=True)
                    ctx_heads.append(
                        jnp.dot(w, vh, preferred_element_type=jnp.float32))
                ctx = jnp.concatenate(ctx_heads, axis=-1)
                outs.append(jnp.dot(ctx, wo, preferred_element_type=jnp.float32))
            return jnp.stack(outs)

        for h in range(N_DEV - 1):
            src = x_ref if h == 0 else xbuf.at[h - 1]
            rdma = pltpu.make_async_remote_copy(
                src_ref=src,
                dst_ref=xbuf.at[h],
                send_sem=ag_send_sems.at[h],
                recv_sem=ag_recv_sems.at[h],
                device_id=(right,),
                device_id_type=pl.DeviceIdType.MESH,
            )
            rdma.start()
            rdma.wait()

        for r in range(N_DEV - 1):
            p = compute_partial(xbuf[r])
            if r == 0:
                rs_send[...] = p
            else:
                rs_send[...] = rs_recv[r - 1] + p
            rdma = pltpu.make_async_remote_copy(
                src_ref=rs_send,
                dst_ref=rs_recv.at[r],
                send_sem=rs_send_sems.at[r],
                recv_sem=rs_recv_sems.at[r],
                device_id=(right,),
                device_id_type=pl.DeviceIdType.MESH,
            )
            rdma.start()
            rdma.wait()

        out_ref[...] = rs_recv[N_DEV - 2] + compute_partial(x_ref[...])

    return pl.pallas_call(
        body,
        out_shape=jax.ShapeDtypeStruct((B_LOC, SQ, D), jnp.float32),
        in_specs=[pl.BlockSpec(memory_space=pltpu.VMEM)] * 8,
        out_specs=pl.BlockSpec(memory_space=pltpu.VMEM),
        scratch_shapes=[
            pltpu.VMEM((N_DEV - 1, B_LOC, SQ, D), jnp.float32),
            pltpu.VMEM((N_DEV - 1, B_LOC, SQ, D), jnp.float32),
            pltpu.VMEM((B_LOC, SQ, D), jnp.float32),
            pltpu.SemaphoreType.DMA((N_DEV - 1,)),
            pltpu.SemaphoreType.DMA((N_DEV - 1,)),
            pltpu.SemaphoreType.DMA((N_DEV - 1,)),
            pltpu.SemaphoreType.DMA((N_DEV - 1,)),
        ],
        compiler_params=pltpu.CompilerParams(collective_id=0),
    )(x, Wq, Wk, Wv, Wo, cos, sin, R)


# baseline (device time: 409095 ns/iter reference)
import jax
import jax.numpy as jnp
import numpy as np
from jax import lax
from jax.experimental import pallas as pl
from jax.experimental.pallas import tpu as pltpu

N_DEV = 4
B_LOC = 2
SQ = 512
D = 1024
H_LOC = 8
DH = 128
SCALE = 0.08838834764831843


def _rope_consts():
    inv = 1.0 / (10000.0 ** (np.arange(0, DH, 2) / DH))
    pos = np.arange(SQ)[:, None] * inv[None, :]
    cos = np.repeat(np.cos(pos), 2, axis=-1).astype(np.float32)
    sin = np.repeat(np.sin(pos), 2, axis=-1).astype(np.float32)
    R = np.zeros((DH, DH), np.float32)
    for k in range(DH // 2):
        R[2 * k + 1, 2 * k] = -1.0
        R[2 * k, 2 * k + 1] = 1.0
    return jnp.asarray(cos), jnp.asarray(sin), jnp.asarray(R)


def kernel(x, Wq, Wk, Wv, Wo):
    cos, sin, R = _rope_consts()

    def body(x_ref, wq_ref, wk_ref, wv_ref, wo_ref, cos_ref, sin_ref, r_ref,
             out_ref, xbuf, rs_recv, rs_send,
             ag_send_sems, ag_recv_sems, rs_send_sems, rs_recv_sems):
        my = lax.axis_index("i")
        left = (my + N_DEV - 1) % N_DEV
        right = (my + 1) % N_DEV

        barrier_sem = pltpu.get_barrier_semaphore()
        for nbr in (left, right):
            pl.semaphore_signal(
                barrier_sem, inc=1,
                device_id=(nbr,), device_id_type=pl.DeviceIdType.MESH,
            )
        pl.semaphore_wait(barrier_sem, 2)

        def rope(t):
            rot = jnp.dot(t, r_ref[...], preferred_element_type=jnp.float32)
            return t * cos_ref[...] + rot * sin_ref[...]

        def accumulate_partial(get_x, get_acc, store):
            for b in range(B_LOC):
                xb = get_x(b)
                q = jnp.dot(xb, wq_ref[...], preferred_element_type=jnp.float32)
                k = jnp.dot(xb, wk_ref[...], preferred_element_type=jnp.float32)
                v = jnp.dot(xb, wv_ref[...], preferred_element_type=jnp.float32)
                acc = get_acc(b)
                for h in range(H_LOC):
                    sl = slice(h * DH, (h + 1) * DH)
                    qh = rope(q[:, sl])
                    kh = rope(k[:, sl])
                    s = lax.dot_general(
                        qh, kh, (((1,), (1,)), ((), ())),
                        preferred_element_type=jnp.float32,
                    ) * SCALE
                    s = s - jnp.max(s, axis=-1, keepdims=True)
                    w = jnp.exp(s)
                    w = w / jnp.sum(w, axis=-1, keepdims=True)
                    ctx = jnp.dot(w, v[:, sl],
                                  preferred_element_type=jnp.float32)
                    acc = acc + jnp.dot(ctx, wo_ref[sl, :],
                                        preferred_element_type=jnp.float32)
                store(b, acc)

        for h in range(N_DEV - 1):
            src = x_ref if h == 0 else xbuf.at[h - 1]
            rdma = pltpu.make_async_remote_copy(
                src_ref=src,
                dst_ref=xbuf.at[h],
                send_sem=ag_send_sems.at[h],
                recv_sem=ag_recv_sems.at[h],
                device_id=(right,),
                device_id_type=pl.DeviceIdType.MESH,
            )
            rdma.start()
            rdma.wait()

        for r in range(N_DEV - 1):
            if r == 0:
                get_acc = lambda b: jnp.zeros((SQ, D), jnp.float32)
            else:
                get_acc = lambda b, _r=r: rs_recv[_r - 1, b]
            accumulate_partial(
                get_x=lambda b, _r=r: xbuf[_r, b],
                get_acc=get_acc,
                store=lambda b, val: rs_send.__setitem__((b,), val),
            )
            dst = rs_recv.at[r] if r < N_DEV - 2 else out_ref
            rdma = pltpu.make_async_remote_copy(
                src_ref=rs_send,
                dst_ref=dst,
                send_sem=rs_send_sems.at[r],
                recv_sem=rs_recv_sems.at[r],
                device_id=(right,),
                device_id_type=pl.DeviceIdType.MESH,
            )
            rdma.start()
            rdma.wait()

        accumulate_partial(
            get_x=lambda b: x_ref[b],
            get_acc=lambda b: out_ref[b],
            store=lambda b, val: out_ref.__setitem__((b,), val),
        )

    return pl.pallas_call(
        body,
        out_shape=jax.ShapeDtypeStruct((B_LOC, SQ, D), jnp.float32),
        in_specs=[pl.BlockSpec(memory_space=pltpu.VMEM)] * 8,
        out_specs=pl.BlockSpec(memory_space=pltpu.VMEM),
        scratch_shapes=[
            pltpu.VMEM((N_DEV - 1, B_LOC, SQ, D), jnp.float32),
            pltpu.VMEM((N_DEV - 2, B_LOC, SQ, D), jnp.float32),
            pltpu.VMEM((B_LOC, SQ, D), jnp.float32),
            pltpu.SemaphoreType.DMA((N_DEV - 1,)),
            pltpu.SemaphoreType.DMA((N_DEV - 1,)),
            pltpu.SemaphoreType.DMA((N_DEV - 1,)),
            pltpu.SemaphoreType.DMA((N_DEV - 1,)),
        ],
        compiler_params=pltpu.CompilerParams(
            collective_id=0, vmem_limit_bytes=100 * 1024 * 1024,
        ),
    )(x, Wq, Wk, Wv, Wo, cos, sin, R)


# device time: 322981 ns/iter; 1.2666x vs baseline; 1.2666x over previous
import jax
import jax.numpy as jnp
import numpy as np
from jax import lax
from jax.experimental import pallas as pl
from jax.experimental.pallas import tpu as pltpu

N_DEV = 4
B_LOC = 2
SQ = 512
D = 1024
H_LOC = 8
DH = 128
SCALE = 0.08838834764831843


def _rope_consts():
    inv = 1.0 / (10000.0 ** (np.arange(0, DH, 2) / DH))
    pos = np.arange(SQ)[:, None] * inv[None, :]
    cos = np.repeat(np.cos(pos), 2, axis=-1).astype(np.float32)
    sin = np.repeat(np.sin(pos), 2, axis=-1).astype(np.float32)
    R = np.zeros((DH, DH), np.float32)
    for k in range(DH // 2):
        R[2 * k + 1, 2 * k] = -1.0
        R[2 * k, 2 * k + 1] = 1.0
    return jnp.asarray(cos), jnp.asarray(sin), jnp.asarray(R)


def kernel(x, Wq, Wk, Wv, Wo):
    cos, sin, R = _rope_consts()

    def body(x_ref, wq_ref, wk_ref, wv_ref, wo_ref, cos_ref, sin_ref, r_ref,
             out_ref, xrecv, precv, psend,
             xsend_sems, xrecv_sems, psend_sems, precv_sems):
        my = lax.axis_index("i")

        barrier_sem = pltpu.get_barrier_semaphore()
        for s in range(N_DEV - 1):
            pl.semaphore_signal(
                barrier_sem, inc=1,
                device_id=((my + 1 + s) % N_DEV,),
                device_id_type=pl.DeviceIdType.MESH,
            )
        pl.semaphore_wait(barrier_sem, N_DEV - 1)

        def rope(t):
            rot = jnp.dot(t, r_ref[...], preferred_element_type=jnp.float32)
            return t * cos_ref[...] + rot * sin_ref[...]

        def accumulate_partial(get_x, get_acc, store):
            for b in range(B_LOC):
                xb = get_x(b)
                q = jnp.dot(xb, wq_ref[...], preferred_element_type=jnp.float32)
                k = jnp.dot(xb, wk_ref[...], preferred_element_type=jnp.float32)
                v = jnp.dot(xb, wv_ref[...], preferred_element_type=jnp.float32)
                acc = get_acc(b)
                for h in range(H_LOC):
                    sl = slice(h * DH, (h + 1) * DH)
                    qh = rope(q[:, sl])
                    kh = rope(k[:, sl])
                    s = lax.dot_general(
                        qh, kh, (((1,), (1,)), ((), ())),
                        preferred_element_type=jnp.float32,
                    ) * SCALE
                    s = s - jnp.max(s, axis=-1, keepdims=True)
                    w = jnp.exp(s)
                    w = w / jnp.sum(w, axis=-1, keepdims=True)
                    ctx = jnp.dot(w, v[:, sl],
                                  preferred_element_type=jnp.float32)
                    acc = acc + jnp.dot(ctx, wo_ref[sl, :],
                                        preferred_element_type=jnp.float32)
                store(b, acc)

        xdesc = [
            pltpu.make_async_remote_copy(
                src_ref=x_ref,
                dst_ref=xrecv.at[2 - s],
                send_sem=xsend_sems.at[s],
                recv_sem=xrecv_sems.at[2 - s],
                device_id=((my + 1 + s) % N_DEV,),
                device_id_type=pl.DeviceIdType.MESH,
            )
            for s in range(N_DEV - 1)
        ]
        pdesc = [
            pltpu.make_async_remote_copy(
                src_ref=psend,
                dst_ref=precv.at[2 - s],
                send_sem=psend_sems.at[s],
                recv_sem=precv_sems.at[2 - s],
                device_id=((my + 1 + s) % N_DEV,),
                device_id_type=pl.DeviceIdType.MESH,
            )
            for s in range(N_DEV - 1)
        ]

        for s in range(N_DEV - 1):
            xdesc[s].start()

        accumulate_partial(
            get_x=lambda b: x_ref[b],
            get_acc=lambda b: jnp.zeros((SQ, D), jnp.float32),
            store=lambda b, val: out_ref.__setitem__((b,), val),
        )

        for i, s in enumerate((0, 2, 1)):
            xdesc[2 - s].wait_recv()
            if i > 0:
                pdesc[prev].wait_send()
            accumulate_partial(
                get_x=lambda b, _s=s: xrecv[_s, b],
                get_acc=lambda b: jnp.zeros((SQ, D), jnp.float32),
                store=lambda b, val: psend.__setitem__((b,), val),
            )
            pdesc[s].start()
            prev = s

        for s in range(N_DEV - 1):
            pdesc[2 - s].wait_recv()
        for b in range(B_LOC):
            out_ref[b] = out_ref[b] + (
                precv[0, b] + (precv[1, b] + precv[2, b]))

        for s in range(N_DEV - 1):
            xdesc[s].wait_send()
        pdesc[prev].wait_send()

    return pl.pallas_call(
        body,
        out_shape=jax.ShapeDtypeStruct((B_LOC, SQ, D), jnp.float32),
        in_specs=[pl.BlockSpec(memory_space=pltpu.VMEM)] * 8,
        out_specs=pl.BlockSpec(memory_space=pltpu.VMEM),
        scratch_shapes=[
            pltpu.VMEM((N_DEV - 1, B_LOC, SQ, D), jnp.float32),
            pltpu.VMEM((N_DEV - 1, B_LOC, SQ, D), jnp.float32),
            pltpu.VMEM((B_LOC, SQ, D), jnp.float32),
            pltpu.SemaphoreType.DMA((N_DEV - 1,)),
            pltpu.SemaphoreType.DMA((N_DEV - 1,)),
            pltpu.SemaphoreType.DMA((N_DEV - 1,)),
            pltpu.SemaphoreType.DMA((N_DEV - 1,)),
        ],
        compiler_params=pltpu.CompilerParams(
            collective_id=0, vmem_limit_bytes=100 * 1024 * 1024,
        ),
    )(x, Wq, Wk, Wv, Wo, cos, sin, R)


# device time: 300209 ns/iter; 1.3627x vs baseline; 1.0759x over previous
import jax
import jax.numpy as jnp
import numpy as np
from jax import lax
from jax.experimental import pallas as pl
from jax.experimental.pallas import tpu as pltpu

N_DEV = 4
B_LOC = 2
SQ = 512
D = 1024
H_LOC = 8
DH = 128
SCALE = 0.08838834764831843


def _rope_consts():
    inv = 1.0 / (10000.0 ** (np.arange(0, DH, 2) / DH))
    pos = np.arange(SQ)[:, None] * inv[None, :]
    cos = np.repeat(np.cos(pos), 2, axis=-1).astype(np.float32)
    sin = np.repeat(np.sin(pos), 2, axis=-1).astype(np.float32)
    R = np.zeros((DH, DH), np.float32)
    for k in range(DH // 2):
        R[2 * k + 1, 2 * k] = -1.0
        R[2 * k, 2 * k + 1] = 1.0
    return jnp.asarray(cos), jnp.asarray(sin), jnp.asarray(R)


def kernel(x, Wq, Wk, Wv, Wo):
    cos, sin, R = _rope_consts()

    def body(x_ref, wq_ref, wk_ref, wv_ref, wo_ref, cos_ref, sin_ref, r_ref,
             out_ref, xrecv, precv, psend,
             xsend_sems, xrecv_sems, psend_sems, precv_sems):
        my = lax.axis_index("i")

        barrier_sem = pltpu.get_barrier_semaphore()
        for s in range(N_DEV - 1):
            pl.semaphore_signal(
                barrier_sem, inc=1,
                device_id=((my + 1 + s) % N_DEV,),
                device_id_type=pl.DeviceIdType.MESH,
            )
        pl.semaphore_wait(barrier_sem, N_DEV - 1)

        def rope(t):
            rot = jnp.dot(t, r_ref[...], preferred_element_type=jnp.float32)
            return t * cos_ref[...] + rot * sin_ref[...]

        def accumulate_partial(get_x, get_acc, store):
            for b in range(B_LOC):
                xb = get_x(b)
                q = jnp.dot(xb, wq_ref[...], preferred_element_type=jnp.float32)
                k = jnp.dot(xb, wk_ref[...], preferred_element_type=jnp.float32)
                v = jnp.dot(xb, wv_ref[...], preferred_element_type=jnp.float32)
                acc = get_acc(b)
                for h in range(H_LOC):
                    sl = slice(h * DH, (h + 1) * DH)
                    qh = rope(q[:, sl])
                    kh = rope(k[:, sl])
                    s = lax.dot_general(
                        qh, kh, (((1,), (1,)), ((), ())),
                        preferred_element_type=jnp.float32,
                    ) * SCALE
                    s = s - jnp.max(s, axis=-1, keepdims=True)
                    w = jnp.exp(s)
                    w = w / jnp.sum(w, axis=-1, keepdims=True)
                    ctx = jnp.dot(w, v[:, sl],
                                  preferred_element_type=jnp.float32)
                    acc = acc + jnp.dot(ctx, wo_ref[sl, :],
                                        preferred_element_type=jnp.float32)
                store(b, acc)

        xdesc = [
            pltpu.make_async_remote_copy(
                src_ref=x_ref,
                dst_ref=xrecv.at[2 - s],
                send_sem=xsend_sems.at[s],
                recv_sem=xrecv_sems.at[2 - s],
                device_id=((my + 1 + s) % N_DEV,),
                device_id_type=pl.DeviceIdType.MESH,
            )
            for s in range(N_DEV - 1)
        ]
        pdesc = [
            pltpu.make_async_remote_copy(
                src_ref=psend,
                dst_ref=precv.at[2 - s],
                send_sem=psend_sems.at[s],
                recv_sem=precv_sems.at[2 - s],
                device_id=((my + 1 + s) % N_DEV,),
                device_id_type=pl.DeviceIdType.MESH,
            )
            for s in range(N_DEV - 1)
        ]

        xdesc[0].start()
        xdesc[2].start()

        accumulate_partial(
            get_x=lambda b: x_ref[b],
            get_acc=lambda b: jnp.zeros((SQ, D), jnp.float32),
            store=lambda b, val: out_ref.__setitem__((b,), val),
        )

        xdesc[1].start()

        for i, s in enumerate((0, 2, 1)):
            xdesc[2 - s].wait_recv()
            if i > 0:
                pdesc[prev].wait_send()
            accumulate_partial(
                get_x=lambda b, _s=s: xrecv[_s, b],
                get_acc=lambda b: jnp.zeros((SQ, D), jnp.float32),
                store=lambda b, val: psend.__setitem__((b,), val),
            )
            pdesc[s].start()
            prev = s

        for k in (2, 0, 1):
            pdesc[2 - k].wait_recv()
            for b in range(B_LOC):
                out_ref[b] = out_ref[b] + precv[k, b]

        for s in range(N_DEV - 1):
            xdesc[s].wait_send()
        pdesc[prev].wait_send()

    return pl.pallas_call(
        body,
        out_shape=jax.ShapeDtypeStruct((B_LOC, SQ, D), jnp.float32),
        in_specs=[pl.BlockSpec(memory_space=pltpu.VMEM)] * 8,
        out_specs=pl.BlockSpec(memory_space=pltpu.VMEM),
        scratch_shapes=[
            pltpu.VMEM((N_DEV - 1, B_LOC, SQ, D), jnp.float32),
            pltpu.VMEM((N_DEV - 1, B_LOC, SQ, D), jnp.float32),
            pltpu.VMEM((B_LOC, SQ, D), jnp.float32),
            pltpu.SemaphoreType.DMA((N_DEV - 1,)),
            pltpu.SemaphoreType.DMA((N_DEV - 1,)),
            pltpu.SemaphoreType.DMA((N_DEV - 1,)),
            pltpu.SemaphoreType.DMA((N_DEV - 1,)),
        ],
        compiler_params=pltpu.CompilerParams(
            collective_id=0, vmem_limit_bytes=100 * 1024 * 1024,
        ),
    )(x, Wq, Wk, Wv, Wo, cos, sin, R)


# device time: 187927 ns/iter; 2.1769x vs baseline; 1.5975x over previous
import jax
import jax.numpy as jnp
import numpy as np
from jax import lax
from jax.experimental import pallas as pl
from jax.experimental.pallas import tpu as pltpu

N_DEV = 4
B_LOC = 2
SQ = 512
D = 1024
H_LOC = 8
DH = 128
SCALE = 0.08838834764831843


def _rope_consts():
    inv = 1.0 / (10000.0 ** (np.arange(0, DH, 2) / DH))
    pos = np.arange(SQ)[:, None] * inv[None, :]
    cos = np.repeat(np.cos(pos), 2, axis=-1).astype(np.float32)
    sin = np.repeat(np.sin(pos), 2, axis=-1).astype(np.float32)
    R = np.zeros((DH, DH), np.float32)
    for k in range(DH // 2):
        R[2 * k + 1, 2 * k] = -1.0
        R[2 * k, 2 * k + 1] = 1.0
    return jnp.asarray(cos), jnp.asarray(sin), jnp.asarray(R)


def kernel(x, Wq, Wk, Wv, Wo):
    cos, sin, R = _rope_consts()

    def body(x_ref, wq_ref, wk_ref, wv_ref, wo_ref, cos_ref, sin_ref, r_ref,
             out_ref, xsend, xrecv, precv, psend,
             xsend_sems, xrecv_sems, psend_sems, precv_sems):
        my = lax.axis_index("i")

        xsend[...] = x_ref[...].astype(jnp.bfloat16)

        barrier_sem = pltpu.get_barrier_semaphore()
        for s in range(N_DEV - 1):
            pl.semaphore_signal(
                barrier_sem, inc=1,
                device_id=((my + 1 + s) % N_DEV,),
                device_id_type=pl.DeviceIdType.MESH,
            )
        pl.semaphore_wait(barrier_sem, N_DEV - 1)

        def rope(t):
            rot = jnp.dot(t, r_ref[...], preferred_element_type=jnp.float32)
            return t * cos_ref[...] + rot * sin_ref[...]

        def accumulate_partial(get_x, get_acc, store):
            for b in range(B_LOC):
                xb = get_x(b)
                q = jnp.dot(xb, wq_ref[...], preferred_element_type=jnp.float32)
                k = jnp.dot(xb, wk_ref[...], preferred_element_type=jnp.float32)
                v = jnp.dot(xb, wv_ref[...], preferred_element_type=jnp.float32)
                acc = get_acc(b)
                for h in range(H_LOC):
                    sl = slice(h * DH, (h + 1) * DH)
                    qh = rope(q[:, sl])
                    kh = rope(k[:, sl])
                    s = lax.dot_general(
                        qh, kh, (((1,), (1,)), ((), ())),
                        preferred_element_type=jnp.float32,
                    ) * SCALE
                    s = s - jnp.max(s, axis=-1, keepdims=True)
                    w = jnp.exp(s)
                    w = w / jnp.sum(w, axis=-1, keepdims=True)
                    ctx = jnp.dot(w, v[:, sl],
                                  preferred_element_type=jnp.float32)
                    acc = acc + jnp.dot(ctx, wo_ref[sl, :],
                                        preferred_element_type=jnp.float32)
                store(b, acc)

        xdesc = [
            pltpu.make_async_remote_copy(
                src_ref=xsend,
                dst_ref=xrecv.at[2 - s],
                send_sem=xsend_sems.at[s],
                recv_sem=xrecv_sems.at[2 - s],
                device_id=((my + 1 + s) % N_DEV,),
                device_id_type=pl.DeviceIdType.MESH,
            )
            for s in range(N_DEV - 1)
        ]
        pdesc = [
            pltpu.make_async_remote_copy(
                src_ref=psend,
                dst_ref=precv.at[2 - s],
                send_sem=psend_sems.at[s],
                recv_sem=precv_sems.at[2 - s],
                device_id=((my + 1 + s) % N_DEV,),
                device_id_type=pl.DeviceIdType.MESH,
            )
            for s in range(N_DEV - 1)
        ]

        xdesc[0].start()
        xdesc[2].start()

        accumulate_partial(
            get_x=lambda b: x_ref[b],
            get_acc=lambda b: jnp.zeros((SQ, D), jnp.float32),
            store=lambda b, val: out_ref.__setitem__((b,), val),
        )

        xdesc[1].start()

        for i, s in enumerate((0, 2, 1)):
            xdesc[2 - s].wait_recv()
            if i > 0:
                pdesc[prev].wait_send()
            accumulate_partial(
                get_x=lambda b, _s=s: xrecv[_s, b].astype(jnp.float32),
                get_acc=lambda b: jnp.zeros((SQ, D), jnp.float32),
                store=lambda b, val: psend.__setitem__(
                    (b,), val.astype(jnp.bfloat16)),
            )
            pdesc[s].start()
            prev = s

        for k in (2, 0, 1):
            pdesc[2 - k].wait_recv()
            for b in range(B_LOC):
                out_ref[b] = out_ref[b] + precv[k, b].astype(jnp.float32)

        for s in range(N_DEV - 1):
            xdesc[s].wait_send()
        pdesc[prev].wait_send()

    return pl.pallas_call(
        body,
        out_shape=jax.ShapeDtypeStruct((B_LOC, SQ, D), jnp.float32),
        in_specs=[pl.BlockSpec(memory_space=pltpu.VMEM)] * 8,
        out_specs=pl.BlockSpec(memory_space=pltpu.VMEM),
        scratch_shapes=[
            pltpu.VMEM((B_LOC, SQ, D), jnp.bfloat16),
            pltpu.VMEM((N_DEV - 1, B_LOC, SQ, D), jnp.bfloat16),
            pltpu.VMEM((N_DEV - 1, B_LOC, SQ, D), jnp.bfloat16),
            pltpu.VMEM((B_LOC, SQ, D), jnp.bfloat16),
            pltpu.SemaphoreType.DMA((N_DEV - 1,)),
            pltpu.SemaphoreType.DMA((N_DEV - 1,)),
            pltpu.SemaphoreType.DMA((N_DEV - 1,)),
            pltpu.SemaphoreType.DMA((N_DEV - 1,)),
        ],
        compiler_params=pltpu.CompilerParams(
            collective_id=0, vmem_limit_bytes=100 * 1024 * 1024,
        ),
    )(x, Wq, Wk, Wv, Wo, cos, sin, R)
